# Initial kernel scaffold; baseline (speedup 1.0000x reference)
import jax
import jax.numpy as jnp
from jax.experimental import pallas as pl


def kernel(x, thresholds, y, gamma, beta):
    raise NotImplementedError("write your pallas kernel here")


# speedup vs baseline: 3890.3301x; 3890.3301x over previous
import functools

import jax
import jax.numpy as jnp
from jax.experimental import pallas as pl

_ROWS_PER_BLOCK = 256


def _asnc_ln_body(t_ref, dy_ref, y0_ref, gamma_ref, beta_ref, x_ref, o_ref,
                  *, n_thresh):
    x = x_ref[...]
    acc = jnp.broadcast_to(y0_ref[...], x.shape)
    for j in range(n_thresh):
        tj = t_ref[j:j + 1, :]
        dj = dy_ref[j:j + 1, :]
        acc = acc + jnp.where(x > tj, dj, jnp.float32(0.0))
    mean = jnp.mean(acc, axis=-1, keepdims=True)
    cen = acc - mean
    var = jnp.mean(cen * cen, axis=-1, keepdims=True)
    inv = jax.lax.rsqrt(var + jnp.float32(1e-5))
    o_ref[...] = cen * inv * gamma_ref[...] + beta_ref[...]


@jax.jit
def kernel(x, thresholds, y, gamma, beta):
    shape = x.shape
    H = shape[-1]
    Km1 = thresholds.shape[1]
    x2 = x.reshape(-1, H)
    N = x2.shape[0]

    t_t = thresholds.T
    dy_t = (y[:, 1:] - y[:, :-1]).T
    y0 = y[:, 0].reshape(1, H)
    gamma2 = gamma.reshape(1, H)
    beta2 = beta.reshape(1, H)

    R = _ROWS_PER_BLOCK
    grid = (N // R,)

    out = pl.pallas_call(
        functools.partial(_asnc_ln_body, n_thresh=Km1),
        grid=grid,
        in_specs=[
            pl.BlockSpec((Km1, H), lambda i: (0, 0)),
            pl.BlockSpec((Km1, H), lambda i: (0, 0)),
            pl.BlockSpec((1, H), lambda i: (0, 0)),
            pl.BlockSpec((1, H), lambda i: (0, 0)),
            pl.BlockSpec((1, H), lambda i: (0, 0)),
            pl.BlockSpec((R, H), lambda i: (i, 0)),
        ],
        out_specs=pl.BlockSpec((R, H), lambda i: (i, 0)),
        out_shape=jax.ShapeDtypeStruct((N, H), x.dtype),
    )(t_t, dy_t, y0, gamma2, beta2, x2)
    return out.reshape(shape)
